# baseline (speedup 1.0000x reference)
import jax
import jax.numpy as jnp
from jax.experimental import pallas as pl
from jax.experimental.pallas import tpu as pltpu

CITY = 100
NE = 200
B = 32
HIST = 16
PRED = 8
IN_DIM = 10
HID = 64
GNN_OUT = 2
E_H = 48
E_OUT = 48
NF = IN_DIM - 1
WIND_MEAN_SPEED = 3.0
WIND_STD_SPEED = 2.0
WIND_MEAN_DIR = 180.0
WIND_STD_DIR = 90.0


def _mm(a, b):
    return jax.lax.dot_general(
        a, b, (((1,), (0,)), ((), ())),
        preferred_element_type=jnp.float32,
        precision=jax.lax.Precision.DEFAULT,
    )


def _fwd_kernel(ft_ref, xn0_cb_ref, xn0_rows_ref, ea_ref,
                src_col_ref, tgt_col_ref, src_row_ref, tgt_row_ref,
                w1_dyn_ref, w1_stat_ref, em_b1_ref, em_w2t_ref, em_b2_ref,
                nm_wt_ref, nm_b_ref, x2h_wt_ref, x2h_b_ref,
                h2h_wt_ref, h2h_b_ref, fo_wt_ref, fo_b_ref,
                out_ref,
                s_x, s_xs, s_xt, s_hp, s_ag, s_ec,
                s_fs, s_ft, s_fc, s_fi, s_p1):
    ea = ea_ref[...]
    mean = jnp.mean(ea, axis=0, keepdims=True)
    var = jnp.sum((ea - mean) ** 2, axis=0, keepdims=True) / (NE - 1)
    attr_norm = (ea - mean) / jnp.sqrt(var)

    iota_g = jax.lax.broadcasted_iota(jnp.int32, (NE, CITY), 1)
    g_src = (src_col_ref[...] == iota_g).astype(jnp.float32)
    g_tgt = (tgt_col_ref[...] == iota_g).astype(jnp.float32)
    iota_s = jax.lax.broadcasted_iota(jnp.int32, (CITY, NE), 0)
    s_mat = ((tgt_row_ref[...] == iota_s).astype(jnp.float32)
             - (src_row_ref[...] == iota_s).astype(jnp.float32))

    econst_e = jnp.concatenate([attr_norm, ea], axis=1)
    s_ec[...] = jnp.broadcast_to(econst_e[:, None, :], (NE, B, 4))
    econst = s_ec[...].reshape(NE * B, 4)
    an_rows = econst[:, 0:2]
    dist_rows = econst[:, 2:3]
    cdir_rows = econst[:, 3:4]

    w1_dyn = w1_dyn_ref[...]
    w1_stat = w1_stat_ref[...]
    em_b1 = em_b1_ref[...]
    em_w2t = em_w2t_ref[...]
    em_b2 = em_b2_ref[...]
    nm_wt = nm_wt_ref[...]
    nm_b = nm_b_ref[...]
    x2h_wt = x2h_wt_ref[...]
    x2h_b = x2h_b_ref[...]
    h2h_wt = h2h_wt_ref[...]
    h2h_b = h2h_b_ref[...]
    fo_wt = fo_wt_ref[...]
    fo_b = fo_b_ref[...]

    for i in range(PRED):
        fi_l = ft_ref[i]
        s_fc[...] = fi_l.reshape(CITY, B, NF)
        s_fi[:, i * NF:(i + 1) * NF] = s_fc[...].reshape(CITY * B, NF)
        s_fs[...] = _mm(g_src, fi_l).reshape(NE, B, NF)
        s_ft[...] = _mm(g_tgt, fi_l).reshape(NE, B, NF)
        fs_rows = s_fs[...].reshape(NE * B, NF)
        ftg_rows = s_ft[...].reshape(NE * B, NF)

        speed = fs_rows[:, 5:6] * WIND_STD_SPEED + WIND_MEAN_SPEED
        direc = fs_rows[:, 6:7] * WIND_STD_DIR + WIND_MEAN_DIR
        theta = jnp.abs(cdir_rows - direc)
        ew_rows = jnp.maximum(3.0 * speed * jnp.cos(theta) / dist_rows,
                              0.0)

        zs = jnp.concatenate([fs_rows, ftg_rows, an_rows, ew_rows],
                             axis=1)
        s_p1[:, i * E_H:(i + 1) * E_H] = _mm(zs, w1_stat) + em_b1

    xn_rows = xn0_rows_ref[...]
    xn_cb = xn0_cb_ref[...]
    hn = jnp.zeros((CITY * B, HID), dtype=jnp.float32)

    for i in range(PRED):
        s_xs[...] = _mm(g_src, xn_cb).reshape(NE, B, 1)
        s_xt[...] = _mm(g_tgt, xn_cb).reshape(NE, B, 1)
        xdyn = jnp.concatenate(
            [s_xs[...].reshape(NE * B, 1), s_xt[...].reshape(NE * B, 1)],
            axis=1)

        h = jax.nn.sigmoid(s_p1[:, i * E_H:(i + 1) * E_H]
                           + _mm(xdyn, w1_dyn))
        h = jax.nn.sigmoid(_mm(h, em_w2t) + em_b2)

        hp = _mm(h, nm_wt)
        s_hp[...] = hp.reshape(NE, B, GNN_OUT)
        hp_l = s_hp[...].reshape(NE, B * GNN_OUT)
        agg_l = _mm(s_mat, hp_l)
        s_ag[...] = agg_l.reshape(CITY, B, GNN_OUT)
        agg_rows = s_ag[...].reshape(CITY * B, GNN_OUT)
        xg = jax.nn.sigmoid(agg_rows + nm_b)

        fi_rows = s_fi[:, i * NF:(i + 1) * NF]
        xf = jnp.concatenate([xg, xn_rows, fi_rows], axis=1)
        gx = _mm(xf, x2h_wt) + x2h_b
        gh = _mm(hn, h2h_wt) + h2h_b
        r = jax.nn.sigmoid(gx[:, :HID] + gh[:, :HID])
        u = jax.nn.sigmoid(gx[:, HID:2 * HID] + gh[:, HID:2 * HID])
        n = jnp.tanh(gx[:, 2 * HID:] + r * gh[:, 2 * HID:])
        hn = n + u * (hn - n)

        xn_rows = _mm(hn, fo_wt) + fo_b
        s_x[...] = xn_rows.reshape(CITY, B, 1)
        xn_cb = s_x[...].reshape(CITY, B)
        out_ref[i] = xn_cb


def kernel(pm25_hist, feature, edge_attr, em_w1, em_b1, em_w2, em_b2,
           nm_w, nm_b, x2h_w, x2h_b, h2h_w, h2h_b, fo_w, fo_b, edge_index):
    ft = jnp.transpose(feature[:, HIST:], (1, 2, 0, 3)).reshape(
        PRED, CITY, B * NF)
    xn0_cb = pm25_hist[:, -1].T
    xn0_rows = xn0_cb.reshape(CITY * B, 1)
    ei = edge_index.astype(jnp.int32)
    src_col = ei[0].reshape(NE, 1)
    tgt_col = ei[1].reshape(NE, 1)
    src_row = ei[0].reshape(1, NE)
    tgt_row = ei[1].reshape(1, NE)

    em_w1t = em_w1.T
    w1_dyn = em_w1t[jnp.array([0, IN_DIM])]
    w1_stat = jnp.concatenate(
        [em_w1t[1:IN_DIM], em_w1t[IN_DIM + 1:2 * IN_DIM],
         em_w1t[2 * IN_DIM:]], axis=0)

    out = pl.pallas_call(
        _fwd_kernel,
        out_shape=jax.ShapeDtypeStruct((PRED, CITY, B), jnp.float32),
        scratch_shapes=[
            pltpu.VMEM((CITY, B, 1), jnp.float32),
            pltpu.VMEM((NE, B, 1), jnp.float32),
            pltpu.VMEM((NE, B, 1), jnp.float32),
            pltpu.VMEM((NE, B, GNN_OUT), jnp.float32),
            pltpu.VMEM((CITY, B, GNN_OUT), jnp.float32),
            pltpu.VMEM((NE, B, 4), jnp.float32),
            pltpu.VMEM((NE, B, NF), jnp.float32),
            pltpu.VMEM((NE, B, NF), jnp.float32),
            pltpu.VMEM((CITY, B, NF), jnp.float32),
            pltpu.VMEM((CITY * B, PRED * NF), jnp.float32),
            pltpu.VMEM((NE * B, PRED * E_H), jnp.float32),
        ],
    )(ft, xn0_cb, xn0_rows, edge_attr,
      src_col, tgt_col, src_row, tgt_row,
      w1_dyn, w1_stat, em_b1.reshape(1, E_H),
      em_w2.T, em_b2.reshape(1, E_OUT),
      nm_w.T, nm_b.reshape(1, GNN_OUT),
      x2h_w.T, x2h_b.reshape(1, 3 * HID),
      h2h_w.T, h2h_b.reshape(1, 3 * HID),
      fo_w.T, fo_b.reshape(1, 1))

    return jnp.transpose(out, (2, 0, 1))[..., None]


# speedup vs baseline: 1.1437x; 1.1437x over previous
import jax
import jax.numpy as jnp
from jax.experimental import pallas as pl
from jax.experimental.pallas import tpu as pltpu

CITY = 100
NE = 200
B = 32
HIST = 16
PRED = 8
IN_DIM = 10
HID = 64
GNN_OUT = 2
E_H = 48
E_OUT = 48
NF = IN_DIM - 1
NV = 16
WIND_MEAN_SPEED = 3.0
WIND_STD_SPEED = 2.0
WIND_MEAN_DIR = 180.0
WIND_STD_DIR = 90.0


def _mm(a, b):
    return jax.lax.dot_general(
        a, b, (((1,), (0,)), ((), ())),
        preferred_element_type=jnp.float32,
        precision=jax.lax.Precision.DEFAULT,
    )


def _fwd_kernel(ft_ref, xn0_cb_ref, xn0_rows_ref, ea_ref,
                src_col_ref, tgt_col_ref, src_row_ref, tgt_row_ref,
                em_w1tp_ref, em_b1_ref, em_w2t_ref, em_b2_ref,
                nm_wt_ref, nm_b_ref, x2h_wt_ref, x2h_b_ref,
                h2h_wt_ref, h2h_b_ref, fo_wt_ref, fo_b_ref,
                out_ref,
                s_x, s_fie, s_gb, s_hp, s_ag, s_ec):
    ea = ea_ref[...]
    mean = jnp.mean(ea, axis=0, keepdims=True)
    var = jnp.sum((ea - mean) ** 2, axis=0, keepdims=True) / (NE - 1)
    attr_norm = (ea - mean) / jnp.sqrt(var)

    iota_g = jax.lax.broadcasted_iota(jnp.int32, (2 * NE, CITY), 1)
    st_col = jnp.concatenate([src_col_ref[...], tgt_col_ref[...]], axis=0)
    g_both = (st_col == iota_g).astype(jnp.float32)
    iota_s = jax.lax.broadcasted_iota(jnp.int32, (CITY, NE), 0)
    s_mat = ((tgt_row_ref[...] == iota_s).astype(jnp.float32)
             - (src_row_ref[...] == iota_s).astype(jnp.float32))

    econst_e = jnp.concatenate([attr_norm, ea], axis=1)
    s_ec[...] = jnp.broadcast_to(econst_e[:, None, :], (NE, B, 4))
    econst = s_ec[...].reshape(NE * B, 4)
    an_rows = econst[:, 0:2]
    dist_rows = econst[:, 2:3]
    cdir_rows = econst[:, 3:4]

    em_w1tp = em_w1tp_ref[...]
    em_b1 = em_b1_ref[...]
    em_w2t = em_w2t_ref[...]
    em_b2 = em_b2_ref[...]
    nm_wt = nm_wt_ref[...]
    nm_b = nm_b_ref[...]
    x2h_wt = x2h_wt_ref[...]
    x2h_b = x2h_b_ref[...]
    h2h_wt = h2h_wt_ref[...]
    h2h_b = h2h_b_ref[...]
    fo_wt = fo_wt_ref[...]
    fo_b = fo_b_ref[...]

    s_fie[:, :, IN_DIM:] = jnp.zeros((CITY, B, NV - IN_DIM), jnp.float32)

    xn_rows = xn0_rows_ref[...]
    xn_cb = xn0_cb_ref[...]
    hn = jnp.zeros((CITY * B, HID), dtype=jnp.float32)

    for i in range(PRED):
        s_fie[:, :, 0:1] = xn_cb.reshape(CITY, B, 1)
        s_fie[:, :, 1:IN_DIM] = ft_ref[i].reshape(CITY, B, NF)
        fie_l = s_fie[...].reshape(CITY, B * NV)

        s_gb[...] = _mm(g_both, fie_l).reshape(2 * NE, B, NV)
        g_rows = s_gb[...].reshape(2 * NE * B, NV)
        gsrc_rows = g_rows[:NE * B]
        gtgt_rows = g_rows[NE * B:]

        speed = gsrc_rows[:, 6:7] * WIND_STD_SPEED + WIND_MEAN_SPEED
        direc = gsrc_rows[:, 7:8] * WIND_STD_DIR + WIND_MEAN_DIR
        theta = jnp.abs(cdir_rows - direc)
        ew_rows = jnp.maximum(3.0 * speed * jnp.cos(theta) / dist_rows,
                              0.0)

        z = jnp.concatenate([gsrc_rows, gtgt_rows, an_rows, ew_rows],
                            axis=1)
        h = jax.nn.sigmoid(_mm(z, em_w1tp) + em_b1)
        h = jax.nn.sigmoid(_mm(h, em_w2t) + em_b2)

        hp = _mm(h, nm_wt)
        s_hp[...] = hp.reshape(NE, B, GNN_OUT)
        hp_l = s_hp[...].reshape(NE, B * GNN_OUT)
        agg_l = _mm(s_mat, hp_l)
        s_ag[...] = agg_l.reshape(CITY, B, GNN_OUT)
        agg_rows = s_ag[...].reshape(CITY * B, GNN_OUT)
        xg = jax.nn.sigmoid(agg_rows + nm_b)

        fie_rows = s_fie[...].reshape(CITY * B, NV)
        xf = jnp.concatenate([xg, fie_rows[:, :IN_DIM]], axis=1)
        gx = _mm(xf, x2h_wt) + x2h_b
        gh = _mm(hn, h2h_wt) + h2h_b
        r = jax.nn.sigmoid(gx[:, :HID] + gh[:, :HID])
        u = jax.nn.sigmoid(gx[:, HID:2 * HID] + gh[:, HID:2 * HID])
        n = jnp.tanh(gx[:, 2 * HID:] + r * gh[:, 2 * HID:])
        hn = n + u * (hn - n)

        xn_rows = _mm(hn, fo_wt) + fo_b
        s_x[...] = xn_rows.reshape(CITY, B, 1)
        xn_cb = s_x[...].reshape(CITY, B)
        out_ref[i] = xn_cb


def kernel(pm25_hist, feature, edge_attr, em_w1, em_b1, em_w2, em_b2,
           nm_w, nm_b, x2h_w, x2h_b, h2h_w, h2h_b, fo_w, fo_b, edge_index):
    ft = jnp.transpose(feature[:, HIST:], (1, 2, 0, 3)).reshape(
        PRED, CITY, B * NF)
    xn0_cb = pm25_hist[:, -1].T
    xn0_rows = xn0_cb.reshape(CITY * B, 1)
    ei = edge_index.astype(jnp.int32)
    src_col = ei[0].reshape(NE, 1)
    tgt_col = ei[1].reshape(NE, 1)
    src_row = ei[0].reshape(1, NE)
    tgt_row = ei[1].reshape(1, NE)

    em_w1t = em_w1.T
    zpad = jnp.zeros((NV - IN_DIM, E_H), jnp.float32)
    em_w1tp = jnp.concatenate(
        [em_w1t[:IN_DIM], zpad, em_w1t[IN_DIM:2 * IN_DIM], zpad,
         em_w1t[2 * IN_DIM:]], axis=0)

    out = pl.pallas_call(
        _fwd_kernel,
        out_shape=jax.ShapeDtypeStruct((PRED, CITY, B), jnp.float32),
        scratch_shapes=[
            pltpu.VMEM((CITY, B, 1), jnp.float32),
            pltpu.VMEM((CITY, B, NV), jnp.float32),
            pltpu.VMEM((2 * NE, B, NV), jnp.float32),
            pltpu.VMEM((NE, B, GNN_OUT), jnp.float32),
            pltpu.VMEM((CITY, B, GNN_OUT), jnp.float32),
            pltpu.VMEM((NE, B, 4), jnp.float32),
        ],
    )(ft, xn0_cb, xn0_rows, edge_attr,
      src_col, tgt_col, src_row, tgt_row,
      em_w1tp, em_b1.reshape(1, E_H),
      em_w2.T, em_b2.reshape(1, E_OUT),
      nm_w.T, nm_b.reshape(1, GNN_OUT),
      x2h_w.T, x2h_b.reshape(1, 3 * HID),
      h2h_w.T, h2h_b.reshape(1, 3 * HID),
      fo_w.T, fo_b.reshape(1, 1))

    return jnp.transpose(out, (2, 0, 1))[..., None]


# speedup vs baseline: 1.1547x; 1.0096x over previous
import jax
import jax.numpy as jnp
from jax.experimental import pallas as pl
from jax.experimental.pallas import tpu as pltpu

CITY = 100
NE = 200
B = 32
HIST = 16
PRED = 8
IN_DIM = 10
HID = 64
GNN_OUT = 2
E_H = 48
E_OUT = 48
NF = IN_DIM - 1
NV = 16
WIND_MEAN_SPEED = 3.0
WIND_STD_SPEED = 2.0
WIND_MEAN_DIR = 180.0
WIND_STD_DIR = 90.0


def _mm(a, b):
    return jax.lax.dot_general(
        a, b, (((1,), (0,)), ((), ())),
        preferred_element_type=jnp.float32,
        precision=jax.lax.Precision.DEFAULT,
    )


def _fwd_kernel(ft_ref, xn0_cb_ref, xn0_rows_ref, ea_ref,
                src_col_ref, tgt_col_ref, src_row_ref, tgt_row_ref,
                w_st_ref, w_ew_ref, w_an_ref, em_b1_ref,
                em_w2t_ref, em_b2_ref,
                nm_wt_ref, nm_b_ref, x2h_wt_ref, x2h_b_ref,
                h2h_wt_ref, h2h_b_ref, fo_wt_ref, fo_b_ref,
                out_ref,
                s_x, s_fie, s_gb, s_hp, s_ag, s_ec):
    ea = ea_ref[...]
    mean = jnp.mean(ea, axis=0, keepdims=True)
    var = jnp.sum((ea - mean) ** 2, axis=0, keepdims=True) / (NE - 1)
    attr_norm = (ea - mean) / jnp.sqrt(var)

    iota_g = jax.lax.broadcasted_iota(jnp.int32, (2 * NE, CITY), 1)
    st_col = jnp.concatenate([src_col_ref[...], tgt_col_ref[...]], axis=0)
    g_both = (st_col == iota_g).astype(jnp.float32)
    iota_s = jax.lax.broadcasted_iota(jnp.int32, (CITY, NE), 0)
    s_mat = ((tgt_row_ref[...] == iota_s).astype(jnp.float32)
             - (src_row_ref[...] == iota_s).astype(jnp.float32))

    econst_e = jnp.concatenate([attr_norm, ea], axis=1)
    s_ec[...] = jnp.broadcast_to(econst_e[:, None, :], (NE, B, 4))
    econst = s_ec[...].reshape(NE * B, 4)
    an_rows = econst[:, 0:2]
    cdir_rows = econst[:, 3:4]
    inv3dist_rows = 3.0 / econst[:, 2:3]

    p1_static = _mm(an_rows, w_an_ref[...]) + em_b1_ref[...]

    w_st = w_st_ref[...]
    w_ew = w_ew_ref[...]
    em_w2t = em_w2t_ref[...]
    em_b2 = em_b2_ref[...]
    nm_wt = nm_wt_ref[...]
    nm_b = nm_b_ref[...]
    x2h_wt = x2h_wt_ref[...]
    x2h_b = x2h_b_ref[...]
    h2h_wt = h2h_wt_ref[...]
    h2h_b = h2h_b_ref[...]
    fo_wt = fo_wt_ref[...]
    fo_b = fo_b_ref[...]

    s_fie[:, :, IN_DIM:] = jnp.zeros((CITY, B, NV - IN_DIM), jnp.float32)

    xn_rows = xn0_rows_ref[...]
    xn_cb = xn0_cb_ref[...]
    hn = jnp.zeros((CITY * B, HID), dtype=jnp.float32)

    for i in range(PRED):
        s_fie[:, :, 0:1] = xn_cb.reshape(CITY, B, 1)
        s_fie[:, :, 1:IN_DIM] = ft_ref[i].reshape(CITY, B, NF)
        fie_l = s_fie[...].reshape(CITY, B * NV)

        s_gb[...] = _mm(g_both, fie_l).reshape(2 * NE, B, NV)
        g_rows = s_gb[...].reshape(2 * NE * B, NV)
        gsrc_rows = g_rows[:NE * B]
        gtgt_rows = g_rows[NE * B:]

        speed = gsrc_rows[:, 6:7] * WIND_STD_SPEED + WIND_MEAN_SPEED
        direc = gsrc_rows[:, 7:8] * WIND_STD_DIR + WIND_MEAN_DIR
        theta = jnp.abs(cdir_rows - direc)
        ew_rows = jnp.maximum(speed * jnp.cos(theta) * inv3dist_rows,
                              0.0)

        q = _mm(g_rows, w_st)
        pre1 = (q[:NE * B, :E_H] + q[NE * B:, E_H:]
                + p1_static + ew_rows * w_ew)
        h = jax.nn.sigmoid(pre1)
        h = jax.nn.sigmoid(_mm(h, em_w2t) + em_b2)

        hp = _mm(h, nm_wt)
        s_hp[...] = hp.reshape(NE, B, GNN_OUT)
        hp_l = s_hp[...].reshape(NE, B * GNN_OUT)
        agg_l = _mm(s_mat, hp_l)
        s_ag[...] = agg_l.reshape(CITY, B, GNN_OUT)
        agg_rows = s_ag[...].reshape(CITY * B, GNN_OUT)
        xg = jax.nn.sigmoid(agg_rows + nm_b)

        fie_rows = s_fie[...].reshape(CITY * B, NV)
        xf = jnp.concatenate([xg, fie_rows[:, :IN_DIM]], axis=1)
        gx = _mm(xf, x2h_wt) + x2h_b
        gh = _mm(hn, h2h_wt) + h2h_b
        r = jax.nn.sigmoid(gx[:, :HID] + gh[:, :HID])
        u = jax.nn.sigmoid(gx[:, HID:2 * HID] + gh[:, HID:2 * HID])
        n = jnp.tanh(gx[:, 2 * HID:] + r * gh[:, 2 * HID:])
        hn = n + u * (hn - n)

        xn_rows = _mm(hn, fo_wt) + fo_b
        s_x[...] = xn_rows.reshape(CITY, B, 1)
        xn_cb = s_x[...].reshape(CITY, B)
        out_ref[i] = xn_cb


def kernel(pm25_hist, feature, edge_attr, em_w1, em_b1, em_w2, em_b2,
           nm_w, nm_b, x2h_w, x2h_b, h2h_w, h2h_b, fo_w, fo_b, edge_index):
    ft = jnp.transpose(feature[:, HIST:], (1, 2, 0, 3)).reshape(
        PRED, CITY, B * NF)
    xn0_cb = pm25_hist[:, -1].T
    xn0_rows = xn0_cb.reshape(CITY * B, 1)
    ei = edge_index.astype(jnp.int32)
    src_col = ei[0].reshape(NE, 1)
    tgt_col = ei[1].reshape(NE, 1)
    src_row = ei[0].reshape(1, NE)
    tgt_row = ei[1].reshape(1, NE)

    em_w1t = em_w1.T
    zpad = jnp.zeros((NV - IN_DIM, E_H), jnp.float32)
    w_src = jnp.concatenate([em_w1t[:IN_DIM], zpad], axis=0)
    w_tgt = jnp.concatenate([em_w1t[IN_DIM:2 * IN_DIM], zpad], axis=0)
    w_st = jnp.concatenate([w_src, w_tgt], axis=1)
    w_an = em_w1t[2 * IN_DIM:2 * IN_DIM + 2]
    w_ew = em_w1t[2 * IN_DIM + 2:]

    out = pl.pallas_call(
        _fwd_kernel,
        out_shape=jax.ShapeDtypeStruct((PRED, CITY, B), jnp.float32),
        scratch_shapes=[
            pltpu.VMEM((CITY, B, 1), jnp.float32),
            pltpu.VMEM((CITY, B, NV), jnp.float32),
            pltpu.VMEM((2 * NE, B, NV), jnp.float32),
            pltpu.VMEM((NE, B, GNN_OUT), jnp.float32),
            pltpu.VMEM((CITY, B, GNN_OUT), jnp.float32),
            pltpu.VMEM((NE, B, 4), jnp.float32),
        ],
    )(ft, xn0_cb, xn0_rows, edge_attr,
      src_col, tgt_col, src_row, tgt_row,
      w_st, w_ew, w_an, em_b1.reshape(1, E_H),
      em_w2.T, em_b2.reshape(1, E_OUT),
      nm_w.T, nm_b.reshape(1, GNN_OUT),
      x2h_w.T, x2h_b.reshape(1, 3 * HID),
      h2h_w.T, h2h_b.reshape(1, 3 * HID),
      fo_w.T, fo_b.reshape(1, 1))

    return jnp.transpose(out, (2, 0, 1))[..., None]


# speedup vs baseline: 1.1731x; 1.0159x over previous
import jax
import jax.numpy as jnp
from jax.experimental import pallas as pl
from jax.experimental.pallas import tpu as pltpu

CITY = 100
NE = 200
B = 32
HIST = 16
PRED = 8
IN_DIM = 10
HID = 64
GNN_OUT = 2
E_H = 48
E_OUT = 48
NF = IN_DIM - 1
WIND_MEAN_SPEED = 3.0
WIND_STD_SPEED = 2.0
WIND_MEAN_DIR = 180.0
WIND_STD_DIR = 90.0


def _mm(a, b):
    return jax.lax.dot_general(
        a, b, (((1,), (0,)), ((), ())),
        preferred_element_type=jnp.float32,
        precision=jax.lax.Precision.DEFAULT,
    )


def _fwd_kernel(ft_ref, xn0_cb_ref, xn0_rows_ref, ea_ref,
                src_col_ref, tgt_col_ref, src_row_ref, tgt_row_ref,
                em_w1t_ref, em_b1_ref, em_w2t_ref, em_b2_ref,
                nm_wt_ref, nm_b_ref, x2h_wt_ref, x2h_b_ref,
                h2h_wt_ref, h2h_b_ref, fo_wt_ref, fo_b_ref,
                out_ref,
                s_x, s_fie, s_gs, s_gt, s_hp, s_ag, s_ec):
    ea = ea_ref[...]
    mean = jnp.mean(ea, axis=0, keepdims=True)
    var = jnp.sum((ea - mean) ** 2, axis=0, keepdims=True) / (NE - 1)
    attr_norm = (ea - mean) / jnp.sqrt(var)

    iota_g = jax.lax.broadcasted_iota(jnp.int32, (NE, CITY), 1)
    g_src = (src_col_ref[...] == iota_g).astype(jnp.float32)
    g_tgt = (tgt_col_ref[...] == iota_g).astype(jnp.float32)
    iota_s = jax.lax.broadcasted_iota(jnp.int32, (CITY, NE), 0)
    s_mat = ((tgt_row_ref[...] == iota_s).astype(jnp.float32)
             - (src_row_ref[...] == iota_s).astype(jnp.float32))

    econst_e = jnp.concatenate([attr_norm, ea], axis=1)
    s_ec[...] = jnp.broadcast_to(econst_e[:, None, :], (NE, B, 4))
    econst = s_ec[...].reshape(NE * B, 4)
    an_rows = econst[:, 0:2]
    dist_rows = econst[:, 2:3]
    cdir_rows = econst[:, 3:4]

    em_w1t = em_w1t_ref[...]
    em_b1 = em_b1_ref[...]
    em_w2t = em_w2t_ref[...]
    em_b2 = em_b2_ref[...]
    nm_wt = nm_wt_ref[...]
    nm_b = nm_b_ref[...]
    x2h_wt = x2h_wt_ref[...]
    x2h_b = x2h_b_ref[...]
    h2h_wt = h2h_wt_ref[...]
    h2h_b = h2h_b_ref[...]
    fo_wt = fo_wt_ref[...]
    fo_b = fo_b_ref[...]

    xn_rows = xn0_rows_ref[...]
    xn_cb = xn0_cb_ref[...]
    hn = jnp.zeros((CITY * B, HID), dtype=jnp.float32)

    for i in range(PRED):
        s_fie[:, :, 0:1] = xn_cb.reshape(CITY, B, 1)
        s_fie[:, :, 1:IN_DIM] = ft_ref[i].reshape(CITY, B, NF)
        fie_l = s_fie[...].reshape(CITY, B * IN_DIM)

        s_gs[...] = _mm(g_src, fie_l).reshape(NE, B, IN_DIM)
        s_gt[...] = _mm(g_tgt, fie_l).reshape(NE, B, IN_DIM)
        gsrc_rows = s_gs[...].reshape(NE * B, IN_DIM)
        gtgt_rows = s_gt[...].reshape(NE * B, IN_DIM)

        speed = gsrc_rows[:, 6:7] * WIND_STD_SPEED + WIND_MEAN_SPEED
        direc = gsrc_rows[:, 7:8] * WIND_STD_DIR + WIND_MEAN_DIR
        theta = jnp.abs(cdir_rows - direc)
        ew_rows = jnp.maximum(3.0 * speed * jnp.cos(theta) / dist_rows,
                              0.0)

        z = jnp.concatenate([gsrc_rows, gtgt_rows, an_rows, ew_rows],
                            axis=1)
        h = jax.nn.sigmoid(_mm(z, em_w1t) + em_b1)
        h = jax.nn.sigmoid(_mm(h, em_w2t) + em_b2)

        hp = _mm(h, nm_wt)
        s_hp[...] = hp.reshape(NE, B, GNN_OUT)
        hp_l = s_hp[...].reshape(NE, B * GNN_OUT)
        agg_l = _mm(s_mat, hp_l)
        s_ag[...] = agg_l.reshape(CITY, B, GNN_OUT)
        agg_rows = s_ag[...].reshape(CITY * B, GNN_OUT)
        xg = jax.nn.sigmoid(agg_rows + nm_b)

        fie_rows = s_fie[...].reshape(CITY * B, IN_DIM)
        xf = jnp.concatenate([xg, fie_rows], axis=1)
        gx = _mm(xf, x2h_wt) + x2h_b
        gh = _mm(hn, h2h_wt) + h2h_b
        r = jax.nn.sigmoid(gx[:, :HID] + gh[:, :HID])
        u = jax.nn.sigmoid(gx[:, HID:2 * HID] + gh[:, HID:2 * HID])
        n = jnp.tanh(gx[:, 2 * HID:] + r * gh[:, 2 * HID:])
        hn = n + u * (hn - n)

        xn_rows = _mm(hn, fo_wt) + fo_b
        s_x[...] = xn_rows.reshape(CITY, B, 1)
        xn_cb = s_x[...].reshape(CITY, B)
        out_ref[i] = xn_cb


def kernel(pm25_hist, feature, edge_attr, em_w1, em_b1, em_w2, em_b2,
           nm_w, nm_b, x2h_w, x2h_b, h2h_w, h2h_b, fo_w, fo_b, edge_index):
    ft = jnp.transpose(feature[:, HIST:], (1, 2, 0, 3)).reshape(
        PRED, CITY, B * NF)
    xn0_cb = pm25_hist[:, -1].T
    xn0_rows = xn0_cb.reshape(CITY * B, 1)
    ei = edge_index.astype(jnp.int32)
    src_col = ei[0].reshape(NE, 1)
    tgt_col = ei[1].reshape(NE, 1)
    src_row = ei[0].reshape(1, NE)
    tgt_row = ei[1].reshape(1, NE)

    out = pl.pallas_call(
        _fwd_kernel,
        out_shape=jax.ShapeDtypeStruct((PRED, CITY, B), jnp.float32),
        scratch_shapes=[
            pltpu.VMEM((CITY, B, 1), jnp.float32),
            pltpu.VMEM((CITY, B, IN_DIM), jnp.float32),
            pltpu.VMEM((NE, B, IN_DIM), jnp.float32),
            pltpu.VMEM((NE, B, IN_DIM), jnp.float32),
            pltpu.VMEM((NE, B, GNN_OUT), jnp.float32),
            pltpu.VMEM((CITY, B, GNN_OUT), jnp.float32),
            pltpu.VMEM((NE, B, 4), jnp.float32),
        ],
    )(ft, xn0_cb, xn0_rows, edge_attr,
      src_col, tgt_col, src_row, tgt_row,
      em_w1.T, em_b1.reshape(1, E_H),
      em_w2.T, em_b2.reshape(1, E_OUT),
      nm_w.T, nm_b.reshape(1, GNN_OUT),
      x2h_w.T, x2h_b.reshape(1, 3 * HID),
      h2h_w.T, h2h_b.reshape(1, 3 * HID),
      fo_w.T, fo_b.reshape(1, 1))

    return jnp.transpose(out, (2, 0, 1))[..., None]
